# baseline (speedup 1.0000x reference)
import functools

import jax
import jax.numpy as jnp
from jax import lax
from jax.experimental import pallas as pl
from jax.experimental.pallas import tpu as pltpu
from jax.experimental.pallas import tpu_sc as plsc

EMBED_DIM = 32


@functools.cache
def _make_gather(n_b: int, n_s: int, n_super: int):
    info = plsc.get_sparse_core_info()
    nc, ns = info.num_cores, info.num_subcores
    nw = nc * ns
    bpw = n_b // nw
    half = bpw // 2
    n_te = EMBED_DIM // 8
    n_tb = half // 128
    assert n_b % (nw * 256) == 0

    mesh = plsc.VectorSubcoreMesh(core_axis_name="c", subcore_axis_name="s")

    @functools.partial(
        pl.kernel,
        mesh=mesh,
        out_type=jax.ShapeDtypeStruct((n_s * EMBED_DIM * n_b,), jnp.float32),
        scratch_types=[
            pltpu.VMEM((n_s, bpw), jnp.int32),
            pltpu.VMEM((n_s * bpw,), jnp.int32),
            pltpu.VMEM((2, half, 128), jnp.float32),
            pltpu.VMEM((2, n_te * n_tb * 8 * 128), jnp.float32),
            [pltpu.SemaphoreType.DMA] * 2,
            [pltpu.SemaphoreType.DMA] * 2,
        ],
        compiler_params=pltpu.CompilerParams(needs_layout_passes=False),
    )
    def gather_kernel(
        idx_hbm, table_hbm, out_hbm, idx_v, idxg_v, rows_v, tbuf_v, gsems, ssems
    ):
        wid = lax.axis_index("s") * nc + lax.axis_index("c")
        b0 = wid * bpw
        pltpu.sync_copy(idx_hbm.at[:, pl.ds(b0, bpw)], idx_v)
        iota = lax.iota(jnp.int32, 16)
        row_ids = [tbl * 128 + j * 16 + iota for tbl in range(n_tb) for j in range(8)]

        def pre_body(u, carry):
            s = u >> 5
            c = (u & 31) * 16
            idxg_v[pl.ds(s * bpw + c, 16)] = idx_v[s, pl.ds(c, 16)] >> 2
            return carry

        lax.fori_loop(0, n_s * (bpw // 16), pre_body, 0)

        def gather_args(s, h):
            return (
                table_hbm.at[idxg_v.at[pl.ds(s * bpw + h * half, half)]],
                rows_v.at[h],
                gsems[h],
            )

        def start_gather(s, h):
            return pltpu.async_copy(*gather_args(s, h))

        def store_copies(s, h):
            return [
                (
                    tbuf_v.at[h, pl.ds(te * (n_tb * 1024), n_tb * 1024)],
                    out_hbm.at[
                        pl.ds(
                            ((s * n_te + te) * (n_b // 128)
                             + wid * 2 * n_tb + h * n_tb) * 1024,
                            n_tb * 1024,
                        )
                    ],
                    ssems[h],
                )
                for te in range(n_te)
            ]

        def transpose(s, h):
            colbase = []
            for k in range(2 * 8):
                rawv = idx_v[s, pl.ds(h * half + k * 16, 16)]
                colbase.append((rawv & 3) * 32)

            def body(g, carry):
                base_g = (g >> 3) * (n_tb * 1024) + (g & 7) * 128
                for tbl in range(n_tb):
                    for j in range(8):
                        k = tbl * 8 + j
                        v = plsc.load_gather(
                            rows_v.at[h], [row_ids[k], colbase[k] + g]
                        )
                        tbuf_v[h, pl.ds(base_g + tbl * 1024 + j * 16, 16)] = v
                return carry

            lax.fori_loop(0, n_te * 8, body, 0)

        start_gather(0, 0)

        def s_body(s, carry):
            s_next = jnp.minimum(s + 1, n_s - 1)

            @pl.when(s >= 1)
            def _():
                for args in store_copies(s - 1, 0):
                    pltpu.make_async_copy(*args).wait()

            g1 = start_gather(s, 1)
            pltpu.make_async_copy(*gather_args(s, 0)).wait()
            transpose(s, 0)
            for args in store_copies(s, 0):
                pltpu.async_copy(*args)

            @pl.when(s >= 1)
            def _():
                for args in store_copies(s - 1, 1):
                    pltpu.make_async_copy(*args).wait()

            start_gather(s_next, 0)
            g1.wait()
            transpose(s, 1)
            for args in store_copies(s, 1):
                pltpu.async_copy(*args)
            return carry

        lax.fori_loop(0, n_s, s_body, 0)
        pltpu.make_async_copy(*gather_args(n_s - 1, 0)).wait()
        for h in range(2):
            for args in store_copies(n_s - 1, h):
                pltpu.make_async_copy(*args).wait()

    return gather_kernel


def kernel(inputs, embed_table):
    b, s = inputs.shape
    idx_t = inputs.T.astype(jnp.int32)
    table2 = embed_table.reshape(-1, 128)
    out_flat = _make_gather(b, s, table2.shape[0])(idx_t, table2)
    out5 = out_flat.reshape(s, EMBED_DIM // 8, b // 128, 8, 128)
    return out5.transpose(2, 4, 0, 1, 3).reshape(b, s, EMBED_DIM)


# speedup vs baseline: 1.2171x; 1.2171x over previous
import functools

import jax
import jax.numpy as jnp
from jax import lax
from jax.experimental import pallas as pl
from jax.experimental.pallas import tpu as pltpu
from jax.experimental.pallas import tpu_sc as plsc

EMBED_DIM = 32


@functools.cache
def _make_gather(n_b: int, n_s: int, n_super: int):
    info = plsc.get_sparse_core_info()
    nc, ns = info.num_cores, info.num_subcores
    nw = nc * ns
    bpw = n_b // nw
    half = bpw // 2
    n_te = EMBED_DIM // 8
    n_tb = half // 128
    assert n_b % (nw * 256) == 0

    mesh = plsc.VectorSubcoreMesh(core_axis_name="c", subcore_axis_name="s")

    @functools.partial(
        pl.kernel,
        mesh=mesh,
        out_type=jax.ShapeDtypeStruct((n_s * EMBED_DIM * n_b,), jnp.float32),
        scratch_types=[
            pltpu.VMEM((n_s, bpw), jnp.int32),
            pltpu.VMEM((n_s * bpw,), jnp.int32),
            pltpu.VMEM((2, half, 128), jnp.float32),
            pltpu.VMEM((2, n_te * n_tb * 8 * 128), jnp.float32),
            [pltpu.SemaphoreType.DMA] * 2,
            [pltpu.SemaphoreType.DMA] * 2,
        ],
        compiler_params=pltpu.CompilerParams(needs_layout_passes=False),
    )
    def gather_kernel(
        idx_hbm, table_hbm, out_hbm, idx_v, idxg_v, rows_v, tbuf_v, gsems, ssems
    ):
        wid = lax.axis_index("s") * nc + lax.axis_index("c")
        b0 = wid * bpw
        pltpu.sync_copy(idx_hbm.at[:, pl.ds(b0, bpw)], idx_v)
        iota = lax.iota(jnp.int32, 16)
        row_ids = [tbl * 128 + j * 16 + iota for tbl in range(n_tb) for j in range(8)]

        def pre_body(u, carry):
            s = u >> 5
            c = (u & 31) * 16
            idxg_v[pl.ds(s * bpw + c, 16)] = idx_v[s, pl.ds(c, 16)] >> 2
            return carry

        lax.fori_loop(0, n_s * (bpw // 16), pre_body, 0)

        def gather_args(s, h):
            return (
                table_hbm.at[idxg_v.at[pl.ds(s * bpw + h * half, half)]],
                rows_v.at[h],
                gsems[h],
            )

        def start_gather(s, h):
            return pltpu.async_copy(*gather_args(s, h))

        def store_copies(s, h):
            return [
                (
                    tbuf_v.at[h, pl.ds(te * (n_tb * 1024), n_tb * 1024)],
                    out_hbm.at[
                        pl.ds(
                            ((s * n_te + te) * (n_b // 128)
                             + wid * 2 * n_tb + h * n_tb) * 1024,
                            n_tb * 1024,
                        )
                    ],
                    ssems[h],
                )
                for te in range(n_te)
            ]

        def transpose(s, h):
            colbase = []
            for k in range(2 * 8):
                rawv = idx_v[s, pl.ds(h * half + k * 16, 16)]
                colbase.append((rawv & 3) * 32)

            @plsc.parallel_loop(0, n_te * 8, unroll=4)
            def body(g):
                base_g = (g >> 3) * (n_tb * 1024) + (g & 7) * 128
                for tbl in range(n_tb):
                    for j in range(8):
                        k = tbl * 8 + j
                        v = plsc.load_gather(
                            rows_v.at[h], [row_ids[k], colbase[k] + g]
                        )
                        tbuf_v[h, pl.ds(base_g + tbl * 1024 + j * 16, 16)] = v

        start_gather(0, 0)

        def s_body(s, carry):
            s_next = jnp.minimum(s + 1, n_s - 1)

            @pl.when(s >= 1)
            def _():
                for args in store_copies(s - 1, 0):
                    pltpu.make_async_copy(*args).wait()

            g1 = start_gather(s, 1)
            pltpu.make_async_copy(*gather_args(s, 0)).wait()
            transpose(s, 0)
            for args in store_copies(s, 0):
                pltpu.async_copy(*args)

            @pl.when(s >= 1)
            def _():
                for args in store_copies(s - 1, 1):
                    pltpu.make_async_copy(*args).wait()

            start_gather(s_next, 0)
            g1.wait()
            transpose(s, 1)
            for args in store_copies(s, 1):
                pltpu.async_copy(*args)
            return carry

        lax.fori_loop(0, n_s, s_body, 0)
        pltpu.make_async_copy(*gather_args(n_s - 1, 0)).wait()
        for h in range(2):
            for args in store_copies(n_s - 1, h):
                pltpu.make_async_copy(*args).wait()

    return gather_kernel


def kernel(inputs, embed_table):
    b, s = inputs.shape
    idx_t = inputs.T.astype(jnp.int32)
    table2 = embed_table.reshape(-1, 128)
    out_flat = _make_gather(b, s, table2.shape[0])(idx_t, table2)
    out5 = out_flat.reshape(s, EMBED_DIM // 8, b // 128, 8, 128)
    return out5.transpose(2, 4, 0, 1, 3).reshape(b, s, EMBED_DIM)


# speedup vs baseline: 1.2201x; 1.0024x over previous
import functools

import jax
import jax.numpy as jnp
from jax import lax
from jax.experimental import pallas as pl
from jax.experimental.pallas import tpu as pltpu
from jax.experimental.pallas import tpu_sc as plsc

EMBED_DIM = 32


@functools.cache
def _make_gather(n_b: int, n_s: int, n_super: int):
    info = plsc.get_sparse_core_info()
    nc, ns = info.num_cores, info.num_subcores
    nw = nc * ns
    bpw = n_b // nw
    half = bpw // 2
    n_te = EMBED_DIM // 8
    n_tb = half // 128
    assert n_b % (nw * 256) == 0

    mesh = plsc.VectorSubcoreMesh(core_axis_name="c", subcore_axis_name="s")

    @functools.partial(
        pl.kernel,
        mesh=mesh,
        out_type=jax.ShapeDtypeStruct((n_s * EMBED_DIM * n_b,), jnp.float32),
        scratch_types=[
            pltpu.VMEM((n_s, bpw), jnp.int32),
            pltpu.VMEM((n_s * bpw,), jnp.int32),
            pltpu.VMEM((2, half, 128), jnp.float32),
            pltpu.VMEM((2, n_te * n_tb * 8 * 128), jnp.float32),
            [pltpu.SemaphoreType.DMA] * 2,
            [pltpu.SemaphoreType.DMA] * 2,
        ],
        compiler_params=pltpu.CompilerParams(needs_layout_passes=False),
    )
    def gather_kernel(
        idx_hbm, table_hbm, out_hbm, idx_v, idxg_v, rows_v, tbuf_v, gsems, ssems
    ):
        wid = lax.axis_index("s") * nc + lax.axis_index("c")
        b0 = wid * bpw
        pltpu.sync_copy(idx_hbm.at[:, pl.ds(b0, bpw)], idx_v)
        iota = lax.iota(jnp.int32, 16)
        row_ids = [tbl * 128 + j * 16 + iota for tbl in range(n_tb) for j in range(8)]

        def pre_body(u, carry):
            s = u >> 5
            c = (u & 31) * 16
            idxg_v[pl.ds(s * bpw + c, 16)] = idx_v[s, pl.ds(c, 16)] >> 2
            return carry

        lax.fori_loop(0, n_s * (bpw // 16), pre_body, 0)

        def gather_args(s, h):
            return (
                table_hbm.at[idxg_v.at[pl.ds(s * bpw + h * half, half)]],
                rows_v.at[h],
                gsems[h],
            )

        def start_gather(s, h):
            return pltpu.async_copy(*gather_args(s, h))

        def store_copies(s, h):
            return [
                (
                    tbuf_v.at[h, pl.ds(te * (n_tb * 1024), n_tb * 1024)],
                    out_hbm.at[
                        pl.ds(
                            ((s * n_te + te) * (n_b // 128)
                             + wid * 2 * n_tb + h * n_tb) * 1024,
                            n_tb * 1024,
                        )
                    ],
                    ssems[h],
                )
                for te in range(n_te)
            ]

        def transpose(s, h):
            colbase = []
            for k in range(2 * 8):
                rawv = idx_v[s, pl.ds(h * half + k * 16, 16)]
                colbase.append((rawv & 3) * 32)

            @plsc.parallel_loop(0, n_te * 8, unroll=8)
            def body(g):
                base_g = (g >> 3) * (n_tb * 1024) + (g & 7) * 128
                for tbl in range(n_tb):
                    for j in range(8):
                        k = tbl * 8 + j
                        v = plsc.load_gather(
                            rows_v.at[h], [row_ids[k], colbase[k] + g]
                        )
                        tbuf_v[h, pl.ds(base_g + tbl * 1024 + j * 16, 16)] = v

        start_gather(0, 0)

        def s_body(s, carry):
            s_next = jnp.minimum(s + 1, n_s - 1)

            @pl.when(s >= 1)
            def _():
                for args in store_copies(s - 1, 0):
                    pltpu.make_async_copy(*args).wait()

            g1 = start_gather(s, 1)
            pltpu.make_async_copy(*gather_args(s, 0)).wait()
            transpose(s, 0)
            for args in store_copies(s, 0):
                pltpu.async_copy(*args)

            @pl.when(s >= 1)
            def _():
                for args in store_copies(s - 1, 1):
                    pltpu.make_async_copy(*args).wait()

            start_gather(s_next, 0)
            g1.wait()
            transpose(s, 1)
            for args in store_copies(s, 1):
                pltpu.async_copy(*args)
            return carry

        lax.fori_loop(0, n_s, s_body, 0)
        pltpu.make_async_copy(*gather_args(n_s - 1, 0)).wait()
        for h in range(2):
            for args in store_copies(n_s - 1, h):
                pltpu.make_async_copy(*args).wait()

    return gather_kernel


def kernel(inputs, embed_table):
    b, s = inputs.shape
    idx_t = inputs.T.astype(jnp.int32)
    table2 = embed_table.reshape(-1, 128)
    out_flat = _make_gather(b, s, table2.shape[0])(idx_t, table2)
    out5 = out_flat.reshape(s, EMBED_DIM // 8, b // 128, 8, 128)
    return out5.transpose(2, 4, 0, 1, 3).reshape(b, s, EMBED_DIM)
